# baseline (speedup 1.0000x reference)
import functools

import jax
import jax.numpy as jnp
import numpy as np
from jax.experimental import pallas as pl

LAST_Q = 64
V_TOPK = 256
S_TOPK = 512
BQ = 128


def _pattern_indices(q, k):
    B, H, S, D = q.shape
    scale = 1.0 / np.sqrt(D)
    qe = q[:, :, -LAST_Q:, :]
    est = jnp.einsum('bhqd,bhkd->bhqk', qe, k) * scale
    rows = jnp.arange(S - LAST_Q, S)[:, None]
    cols = jnp.arange(S)[None, :]
    causal_e = cols <= rows
    est = jnp.where(causal_e[None, None], est, -jnp.inf)
    est = jax.nn.softmax(est, axis=-1)

    vert = est.sum(axis=2)
    _, v_idx = jax.lax.top_k(vert, V_TOPK)

    est_rev = jnp.where(causal_e[None, None], est, 0.0)[..., ::-1]
    w = S + LAST_Q
    padded = jnp.pad(est_rev, ((0, 0), (0, 0), (0, 0), (0, LAST_Q)))
    flat = padded.reshape(B, H, LAST_Q * w)[:, :, :LAST_Q * (w - 1)]
    shifted = flat.reshape(B, H, LAST_Q, w - 1)
    slash = shifted.sum(axis=2)[..., LAST_Q - 1:LAST_Q - 1 + S]
    _, s_idx = jax.lax.top_k(slash, S_TOPK)
    return v_idx, s_idx


def _flash_body(q_ref, k_ref, v_ref, vb_ref, st_ref, o_ref, *, scale):
    i = pl.program_id(1)
    qb = (q_ref[0] * scale).astype(jnp.bfloat16)

    def body(j, carry):
        acc, m, l = carry
        kj = k_ref[0, pl.ds(j * BQ, BQ), :]
        vj = v_ref[0, pl.ds(j * BQ, BQ), :]
        s = jax.lax.dot_general(qb, kj, (((1,), (1,)), ((), ())),
                                preferred_element_type=jnp.float32)
        row_id = i * BQ + jax.lax.broadcasted_iota(jnp.int32, (BQ, BQ), 0)
        col_id = j * BQ + jax.lax.broadcasted_iota(jnp.int32, (BQ, BQ), 1)
        causal = col_id <= row_id
        st = st_ref[0, pl.ds(i - j, 1)][0]
        vbj = vb_ref[0, 0, pl.ds(j * BQ, BQ)]
        mask = causal & ((vbj[None, :] > 0.0) | (st > 0.0))
        s = jnp.where(mask, s, -1e9)
        m_new = jnp.maximum(m, jnp.max(s, axis=1, keepdims=True))
        alpha = jnp.exp(m - m_new)
        p = jnp.exp(s - m_new).astype(jnp.bfloat16)
        l = l * alpha + jnp.sum(p.astype(jnp.float32), axis=1, keepdims=True)
        acc = acc * alpha + jax.lax.dot_general(
            p, vj, (((1,), (0,)), ((), ())), preferred_element_type=jnp.float32)
        return acc, m_new, l

    D = q_ref.shape[-1]
    acc0 = jnp.zeros((BQ, D), jnp.float32)
    m0 = jnp.full((BQ, 1), -jnp.inf, jnp.float32)
    l0 = jnp.zeros((BQ, 1), jnp.float32)
    acc, m, l = jax.lax.fori_loop(0, i + 1, body, (acc0, m0, l0))
    o_ref[0] = acc / l


def _flash(qh, kh, vh, v_bool, slash_t, *, interpret=False):
    H, S, D = qh.shape
    nt = S // BQ
    scale = 1.0 / np.sqrt(D)
    kb = kh.astype(jnp.bfloat16)
    vb16 = vh.astype(jnp.bfloat16)
    body = functools.partial(_flash_body, scale=scale)
    return pl.pallas_call(
        body,
        grid=(H, S // BQ),
        in_specs=[
            pl.BlockSpec((1, BQ, D), lambda h, i: (h, i, 0)),
            pl.BlockSpec((1, S, D), lambda h, i: (h, 0, 0)),
            pl.BlockSpec((1, S, D), lambda h, i: (h, 0, 0)),
            pl.BlockSpec((1, 1, S), lambda h, i: (h, 0, 0)),
            pl.BlockSpec((1, nt, BQ, BQ), lambda h, i: (h, 0, 0, 0)),
        ],
        out_specs=pl.BlockSpec((1, BQ, D), lambda h, i: (h, i, 0)),
        out_shape=jax.ShapeDtypeStruct((H, S, D), jnp.float32),
        interpret=interpret,
    )(qh, kb, vb16, v_bool, slash_t)


def kernel(q, k, v):
    B, H, S, D = q.shape
    v_idx, s_idx = _pattern_indices(q, k)

    s_r = jnp.arange(S)
    v_bool = (v_idx[..., None] == s_r).any(axis=-2).astype(jnp.float32)
    s_bool = (s_idx[..., None] == s_r).any(axis=-2).astype(jnp.float32)

    nt = S // BQ
    t_idx = (jnp.arange(nt)[:, None, None] * BQ
             + jnp.arange(BQ)[None, :, None] - jnp.arange(BQ)[None, None, :])
    t_idx = jnp.clip(t_idx, 0, S - 1)
    slash_t = s_bool[0][:, t_idx]

    out = _flash(q[0], k[0], v[0], v_bool[0][:, None, :], slash_t)
    out = out[None]

    qmin = jnp.minimum(v_idx.min(-1), s_idx.min(-1))
    mean_v = jnp.mean(v, axis=2, keepdims=True)
    empty = s_r[None, None, :, None] < qmin[:, :, None, None]
    return jnp.where(empty, mean_v, out)


# speedup vs baseline: 1.1974x; 1.1974x over previous
import functools

import jax
import jax.numpy as jnp
import numpy as np
from jax.experimental import pallas as pl
from jax.experimental.pallas import tpu as pltpu

LAST_Q = 64
V_TOPK = 256
S_TOPK = 512
BQ = 128
NEG = -1e9


def _pattern_indices(q, k):
    B, H, S, D = q.shape
    scale = 1.0 / np.sqrt(D)
    qe = q[:, :, -LAST_Q:, :]
    est = jnp.einsum('bhqd,bhkd->bhqk', qe, k) * scale
    rows = jnp.arange(S - LAST_Q, S)[:, None]
    cols = jnp.arange(S)[None, :]
    causal_e = cols <= rows
    est = jnp.where(causal_e[None, None], est, -jnp.inf)
    est = jax.nn.softmax(est, axis=-1)

    vert = est.sum(axis=2)
    _, v_idx = jax.lax.top_k(vert, V_TOPK)

    est_rev = jnp.where(causal_e[None, None], est, 0.0)[..., ::-1]
    w = S + LAST_Q
    padded = jnp.pad(est_rev, ((0, 0), (0, 0), (0, 0), (0, LAST_Q)))
    flat = padded.reshape(B, H, LAST_Q * w)[:, :, :LAST_Q * (w - 1)]
    shifted = flat.reshape(B, H, LAST_Q, w - 1)
    slash = shifted.sum(axis=2)[..., LAST_Q - 1:LAST_Q - 1 + S]
    _, s_idx = jax.lax.top_k(slash, S_TOPK)
    return v_idx, s_idx


def _flash_body(q_ref, k_ref, v_ref, vb_ref, st_ref, o_ref, s_scr, *, scale, nt):
    i = pl.program_id(1)
    qb = (q_ref[0] * scale).astype(jnp.bfloat16)
    causal_bias = jnp.where(
        jax.lax.broadcasted_iota(jnp.int32, (BQ, BQ), 1)
        <= jax.lax.broadcasted_iota(jnp.int32, (BQ, BQ), 0), 0.0, NEG)

    def score_tile(j):
        kj = k_ref[0, pl.ds(j * BQ, BQ), :]
        s = jax.lax.dot_general(qb, kj, (((1,), (1,)), ((), ())),
                                preferred_element_type=jnp.float32)
        st = st_ref[0, pl.ds(jnp.maximum(i - j, 0), 1)][0]
        vbj = vb_ref[0, 0, pl.ds(j * BQ, BQ)]
        bias = jnp.maximum(st, vbj[None, :])
        diag = (j == i).astype(jnp.float32)
        phantom = (j > i).astype(jnp.float32)
        s = s + bias + causal_bias * diag + NEG * phantom
        s_scr[pl.ds(j, 1)] = s[None]
        return jnp.max(s, axis=1, keepdims=True)

    npairs = (i + 2) // 2

    def pass1(p, m):
        m0 = score_tile(2 * p)
        m1 = score_tile(2 * p + 1)
        return jnp.maximum(m, jnp.maximum(m0, m1))

    m = jax.lax.fori_loop(0, npairs, pass1,
                          jnp.full((BQ, 1), NEG, jnp.float32))

    def acc_tile(j, m):
        p = jnp.exp(s_scr[pl.ds(j, 1)][0] - m)
        lj = jnp.sum(p, axis=1, keepdims=True)
        vj = v_ref[0, pl.ds(j * BQ, BQ), :]
        aj = jax.lax.dot_general(p.astype(jnp.bfloat16), vj,
                                 (((1,), (0,)), ((), ())),
                                 preferred_element_type=jnp.float32)
        return lj, aj

    def pass2(p, carry):
        acc, l = carry
        l0, a0 = acc_tile(2 * p, m)
        l1, a1 = acc_tile(2 * p + 1, m)
        return acc + a0 + a1, l + l0 + l1

    D = q_ref.shape[-1]
    acc, l = jax.lax.fori_loop(
        0, npairs, pass2,
        (jnp.zeros((BQ, D), jnp.float32), jnp.zeros((BQ, 1), jnp.float32)))
    o_ref[0] = acc / l


def _flash(qh, kh, vh, v_bias, slash_bias, *, interpret=False):
    H, S, D = qh.shape
    nt = S // BQ
    scale = 1.0 / np.sqrt(D)
    kb = kh.astype(jnp.bfloat16)
    vb16 = vh.astype(jnp.bfloat16)
    body = functools.partial(_flash_body, scale=scale, nt=nt)
    return pl.pallas_call(
        body,
        grid=(H, nt),
        in_specs=[
            pl.BlockSpec((1, BQ, D), lambda h, i: (h, i, 0)),
            pl.BlockSpec((1, S, D), lambda h, i: (h, 0, 0)),
            pl.BlockSpec((1, S, D), lambda h, i: (h, 0, 0)),
            pl.BlockSpec((1, 1, S), lambda h, i: (h, 0, 0)),
            pl.BlockSpec((1, nt, BQ, BQ), lambda h, i: (h, 0, 0, 0)),
        ],
        out_specs=pl.BlockSpec((1, BQ, D), lambda h, i: (h, i, 0)),
        out_shape=jax.ShapeDtypeStruct((H, S, D), jnp.float32),
        scratch_shapes=[pltpu.VMEM((nt + 1, BQ, BQ), jnp.float32)],
        interpret=interpret,
    )(qh, kb, vb16, v_bias, slash_bias)


def kernel(q, k, v):
    B, H, S, D = q.shape
    v_idx, s_idx = _pattern_indices(q, k)

    s_r = jnp.arange(S)
    v_bias = jnp.where((v_idx[..., None] == s_r).any(axis=-2), 0.0, NEG)
    s_bias = jnp.where((s_idx[..., None] == s_r).any(axis=-2), 0.0, NEG)

    nt = S // BQ
    t_idx = (jnp.arange(nt)[:, None, None] * BQ
             + jnp.arange(BQ)[None, :, None] - jnp.arange(BQ)[None, None, :])
    t_idx = jnp.clip(t_idx, 0, S - 1)
    slash_bias = s_bias[0][:, t_idx]

    out = _flash(q[0], k[0], v[0], v_bias[0][:, None, :], slash_bias)
    out = out[None]

    qmin = jnp.minimum(v_idx.min(-1), s_idx.min(-1))
    mean_v = jnp.mean(v, axis=2, keepdims=True)
    empty = s_r[None, None, :, None] < qmin[:, :, None, None]
    return jnp.where(empty, mean_v, out)
